# Initial kernel scaffold; baseline (speedup 1.0000x reference)
import jax
import jax.numpy as jnp
from jax.experimental import pallas as pl


def kernel(x, edge_index, W1, b1, g1, be1, m1, v1, W2, b2, g2, be2, m2, v2, W3, b3):
    raise NotImplementedError("write your pallas kernel here")


# speedup vs baseline: 3.3674x; 3.3674x over previous
import functools

import jax
import jax.numpy as jnp
from jax import lax
from jax.experimental import pallas as pl
from jax.experimental.pallas import tpu as pltpu
from jax.experimental.pallas import tpu_sc as plsc

N_NODES = 10000
D = 128
N_EDGES = 320000
BN_EPS = 1e-5

NC = 2
NS = 16
NW = NC * NS

CHUNK = 128
K = 80
E_PAD = NW * K * CHUNK
ROWS_PER_TILE = 632
LAST_ROWS = N_NODES - 15 * ROWS_PER_TILE
N_PAD = NS * ROWS_PER_TILE


def _sc_aggregate(x, src2, dst2):
    mesh = plsc.VectorSubcoreMesh(core_axis_name="c", subcore_axis_name="s")

    @functools.partial(
        pl.kernel,
        mesh=mesh,
        out_type=jax.ShapeDtypeStruct((NC, N_PAD, D), jnp.float32),
        scratch_types=[
            pltpu.VMEM((K, CHUNK), jnp.int32),
            pltpu.VMEM((K, CHUNK), jnp.int32),
            pltpu.VMEM((CHUNK, D), jnp.float32),
            pltpu.VMEM_SHARED((N_PAD, D), jnp.float32),
            pltpu.SemaphoreType.DMA,
        ],
    )
    def agg_kernel(x_hbm, src_hbm, dst_hbm, out_hbm, src_v, dst_v, buf, acc, sem):
        cid = lax.axis_index("c")
        sid = lax.axis_index("s")
        wid = cid * NS + sid

        row0 = sid * ROWS_PER_TILE

        @pl.when(sid < NS - 1)
        def _():
            pltpu.sync_copy(x_hbm.at[pl.ds(row0, ROWS_PER_TILE)],
                            acc.at[pl.ds(row0, ROWS_PER_TILE)])

        @pl.when(sid == NS - 1)
        def _():
            pltpu.sync_copy(x_hbm.at[pl.ds((NS - 1) * ROWS_PER_TILE, LAST_ROWS)],
                            acc.at[pl.ds((NS - 1) * ROWS_PER_TILE, LAST_ROWS)])

        plsc.subcore_barrier()

        base = wid * K
        pltpu.sync_copy(src_hbm.at[pl.ds(base, K)], src_v)
        pltpu.sync_copy(dst_hbm.at[pl.ds(base, K)], dst_v)

        def body(j, carry):
            pltpu.async_copy(x_hbm.at[src_v.at[j]], buf, sem).wait()
            pltpu.sync_copy(buf, acc.at[dst_v.at[j]], add=True)
            return carry

        lax.fori_loop(0, K, body, 0)
        plsc.subcore_barrier()

        pltpu.sync_copy(acc.at[pl.ds(row0, ROWS_PER_TILE)],
                        out_hbm.at[cid, pl.ds(row0, ROWS_PER_TILE)])

    return agg_kernel(x, src2, dst2)


BR = 1000


def _mlp_body(p_ref, x_ref, w1_ref, c1_ref, w2_ref, c2_ref, w3_ref, c3_ref, o_ref):
    h = p_ref[0] + p_ref[1] - x_ref[...]
    h = jnp.maximum(jnp.dot(h, w1_ref[...], preferred_element_type=jnp.float32)
                    + c1_ref[...], 0.0)
    h = jnp.maximum(jnp.dot(h, w2_ref[...], preferred_element_type=jnp.float32)
                    + c2_ref[...], 0.0)
    o_ref[...] = (jnp.dot(h, w3_ref[...], preferred_element_type=jnp.float32)
                  + c3_ref[...])


def _tc_mlp(parts, x, w1, c1, w2, c2, w3, c3):
    grid = N_NODES // BR
    return pl.pallas_call(
        _mlp_body,
        grid=(grid,),
        in_specs=[
            pl.BlockSpec((NC, BR, D), lambda i: (0, i, 0)),
            pl.BlockSpec((BR, D), lambda i: (i, 0)),
            pl.BlockSpec((D, D), lambda i: (0, 0)),
            pl.BlockSpec((1, D), lambda i: (0, 0)),
            pl.BlockSpec((D, D), lambda i: (0, 0)),
            pl.BlockSpec((1, D), lambda i: (0, 0)),
            pl.BlockSpec((D, D), lambda i: (0, 0)),
            pl.BlockSpec((1, D), lambda i: (0, 0)),
        ],
        out_specs=pl.BlockSpec((BR, D), lambda i: (i, 0)),
        out_shape=jax.ShapeDtypeStruct((N_NODES, D), jnp.float32),
    )(parts, x, w1, c1, w2, c2, w3, c3)


def kernel(x, edge_index, W1, b1, g1, be1, m1, v1, W2, b2, g2, be2, m2, v2, W3, b3):
    src = edge_index[0].astype(jnp.int32)
    dst = edge_index[1].astype(jnp.int32)
    pad = E_PAD - N_EDGES
    src_p = jnp.concatenate([src, jnp.zeros((pad,), jnp.int32)])
    dst_p = jnp.concatenate([dst, jnp.full((pad,), N_NODES, jnp.int32)])
    src2 = src_p.reshape(NW * K, CHUNK)
    dst2 = dst_p.reshape(NW * K, CHUNK)

    parts = _sc_aggregate(x, src2, dst2)

    s1 = g1 * lax.rsqrt(v1 + BN_EPS)
    w1 = W1 * s1[None, :]
    c1 = (b1 * s1 + be1 - m1 * s1)[None, :]
    s2 = g2 * lax.rsqrt(v2 + BN_EPS)
    w2 = W2 * s2[None, :]
    c2 = (b2 * s2 + be2 - m2 * s2)[None, :]
    c3 = b3[None, :]

    return _tc_mlp(parts, x, w1, c1, w2, c2, W3, c3)


# speedup vs baseline: 3.4752x; 1.0320x over previous
import functools

import jax
import jax.numpy as jnp
from jax import lax
from jax.experimental import pallas as pl
from jax.experimental.pallas import tpu as pltpu
from jax.experimental.pallas import tpu_sc as plsc

N_NODES = 10000
D = 128
N_EDGES = 320000
BN_EPS = 1e-5

NC = 2
NS = 16
NW = NC * NS

CHUNK = 128
K = 80
E_PAD = NW * K * CHUNK
ROWS_PER_TILE = 632
LAST_ROWS = N_NODES - 15 * ROWS_PER_TILE
N_PAD = NS * ROWS_PER_TILE
IDX_SHIFT = 14
IDX_MASK = (1 << IDX_SHIFT) - 1


def _sc_aggregate(x, combo2):
    mesh = plsc.VectorSubcoreMesh(core_axis_name="c", subcore_axis_name="s")

    @functools.partial(
        pl.kernel,
        mesh=mesh,
        out_type=jax.ShapeDtypeStruct((NC, N_PAD, D), jnp.float32),
        scratch_types=[
            pltpu.VMEM((K, CHUNK), jnp.int32),
            pltpu.VMEM((CHUNK,), jnp.int32),
            pltpu.VMEM((CHUNK,), jnp.int32),
            pltpu.VMEM((CHUNK,), jnp.int32),
            pltpu.VMEM((CHUNK,), jnp.int32),
            pltpu.VMEM((CHUNK, D), jnp.float32),
            pltpu.VMEM((CHUNK, D), jnp.float32),
            pltpu.VMEM_SHARED((N_PAD, D), jnp.float32),
            pltpu.SemaphoreType.DMA,
            pltpu.SemaphoreType.DMA,
        ],
    )
    def agg_kernel(x_hbm, combo_hbm, out_hbm, combo_v, s0, s1, d0, d1,
                   b0, b1, acc, g0, g1):
        bufs = (b0, b1)
        srcs = (s0, s1)
        dsts = (d0, d1)
        gsems = (g0, g1)
        cid = lax.axis_index("c")
        sid = lax.axis_index("s")
        wid = cid * NS + sid

        row0 = sid * ROWS_PER_TILE

        @pl.when(sid < NS - 1)
        def _():
            pltpu.sync_copy(x_hbm.at[pl.ds(row0, ROWS_PER_TILE)],
                            acc.at[pl.ds(row0, ROWS_PER_TILE)])

        @pl.when(sid == NS - 1)
        def _():
            pltpu.sync_copy(x_hbm.at[pl.ds((NS - 1) * ROWS_PER_TILE, LAST_ROWS)],
                            acc.at[pl.ds((NS - 1) * ROWS_PER_TILE, LAST_ROWS)])

        plsc.subcore_barrier()

        pltpu.sync_copy(combo_hbm.at[pl.ds(wid * K, K)], combo_v)

        def unpack_idx(c, slot):
            for i in range(CHUNK // 16):
                w = combo_v[c, pl.ds(i * 16, 16)]
                srcs[slot][pl.ds(i * 16, 16)] = w & IDX_MASK
                dsts[slot][pl.ds(i * 16, 16)] = lax.shift_right_logical(
                    w, IDX_SHIFT)

        def gstart(b):
            pltpu.async_copy(x_hbm.at[srcs[b]], bufs[b], gsems[b])

        def gwait(b):
            pltpu.make_async_copy(x_hbm.at[srcs[b]], bufs[b], gsems[b]).wait()

        def body(m, carry):
            t0 = m * 2
            unpack_idx(t0, 0)
            gstart(0)
            unpack_idx(t0 + 1, 1)
            gstart(1)
            gwait(0)
            pltpu.sync_copy(bufs[0], acc.at[dsts[0]], add=True)
            gwait(1)
            pltpu.sync_copy(bufs[1], acc.at[dsts[1]], add=True)
            return carry

        lax.fori_loop(0, K // 2, body, 0)

        plsc.subcore_barrier()

        pltpu.sync_copy(acc.at[pl.ds(row0, ROWS_PER_TILE)],
                        out_hbm.at[cid, pl.ds(row0, ROWS_PER_TILE)])

    return agg_kernel(x, combo2)


BR = 1000


def _mlp_body(p_ref, x_ref, w1_ref, c1_ref, w2_ref, c2_ref, w3_ref, c3_ref, o_ref):
    h = p_ref[0] + p_ref[1] - x_ref[...]
    h = jnp.maximum(jnp.dot(h, w1_ref[...], preferred_element_type=jnp.float32)
                    + c1_ref[...], 0.0)
    h = jnp.maximum(jnp.dot(h, w2_ref[...], preferred_element_type=jnp.float32)
                    + c2_ref[...], 0.0)
    o_ref[...] = (jnp.dot(h, w3_ref[...], preferred_element_type=jnp.float32)
                  + c3_ref[...])


def _tc_mlp(parts, x, w1, c1, w2, c2, w3, c3):
    grid = N_NODES // BR
    return pl.pallas_call(
        _mlp_body,
        grid=(grid,),
        in_specs=[
            pl.BlockSpec((NC, BR, D), lambda i: (0, i, 0)),
            pl.BlockSpec((BR, D), lambda i: (i, 0)),
            pl.BlockSpec((D, D), lambda i: (0, 0)),
            pl.BlockSpec((1, D), lambda i: (0, 0)),
            pl.BlockSpec((D, D), lambda i: (0, 0)),
            pl.BlockSpec((1, D), lambda i: (0, 0)),
            pl.BlockSpec((D, D), lambda i: (0, 0)),
            pl.BlockSpec((1, D), lambda i: (0, 0)),
        ],
        out_specs=pl.BlockSpec((BR, D), lambda i: (i, 0)),
        out_shape=jax.ShapeDtypeStruct((N_NODES, D), jnp.float32),
    )(parts, x, w1, c1, w2, c2, w3, c3)


def kernel(x, edge_index, W1, b1, g1, be1, m1, v1, W2, b2, g2, be2, m2, v2, W3, b3):
    src = edge_index[0].astype(jnp.int32)
    dst = edge_index[1].astype(jnp.int32)
    combo = jnp.bitwise_or(jnp.left_shift(dst, IDX_SHIFT), src)
    pad = E_PAD - N_EDGES
    combo_p = jnp.concatenate(
        [combo, jnp.full((pad,), N_NODES << IDX_SHIFT, jnp.int32)])
    combo2 = combo_p.reshape(NW * K, CHUNK)

    parts = _sc_aggregate(x, combo2)

    s1 = g1 * lax.rsqrt(v1 + BN_EPS)
    w1 = W1 * s1[None, :]
    c1 = (b1 * s1 + be1 - m1 * s1)[None, :]
    s2 = g2 * lax.rsqrt(v2 + BN_EPS)
    w2 = W2 * s2[None, :]
    c2 = (b2 * s2 + be2 - m2 * s2)[None, :]
    c3 = b3[None, :]

    return _tc_mlp(parts, x, w1, c1, w2, c2, W3, c3)


# speedup vs baseline: 3.4777x; 1.0007x over previous
import functools

import jax
import jax.numpy as jnp
from jax import lax
from jax.experimental import pallas as pl
from jax.experimental.pallas import tpu as pltpu
from jax.experimental.pallas import tpu_sc as plsc

N_NODES = 10000
D = 128
N_EDGES = 320000
BN_EPS = 1e-5

NC = 2
NS = 16
NW = NC * NS

CHUNK = 128
K = 80
E_PAD = NW * K * CHUNK
ROWS_PER_TILE = 632
LAST_ROWS = N_NODES - 15 * ROWS_PER_TILE
N_PAD = NS * ROWS_PER_TILE
IDX_SHIFT = 14
IDX_MASK = (1 << IDX_SHIFT) - 1


def _sc_aggregate(x, combo2):
    mesh = plsc.VectorSubcoreMesh(core_axis_name="c", subcore_axis_name="s")

    @functools.partial(
        pl.kernel,
        mesh=mesh,
        out_type=jax.ShapeDtypeStruct((NC, N_PAD, D), jnp.float32),
        scratch_types=[
            pltpu.VMEM((K, CHUNK), jnp.int32),
            pltpu.VMEM((CHUNK,), jnp.int32),
            pltpu.VMEM((CHUNK,), jnp.int32),
            pltpu.VMEM((CHUNK,), jnp.int32),
            pltpu.VMEM((CHUNK,), jnp.int32),
            pltpu.VMEM((CHUNK, D), jnp.float32),
            pltpu.VMEM((CHUNK, D), jnp.float32),
            pltpu.VMEM_SHARED((N_PAD, D), jnp.float32),
            pltpu.SemaphoreType.DMA,
            pltpu.SemaphoreType.DMA,
        ],
    )
    def agg_kernel(x_hbm, combo_hbm, out_hbm, combo_v, s0, s1, d0, d1,
                   b0, b1, acc, g0, g1):
        bufs = (b0, b1)
        srcs = (s0, s1)
        dsts = (d0, d1)
        gsems = (g0, g1)
        cid = lax.axis_index("c")
        sid = lax.axis_index("s")
        wid = cid * NS + sid

        row0 = sid * ROWS_PER_TILE

        @pl.when(sid < NS - 1)
        def _():
            pltpu.sync_copy(x_hbm.at[pl.ds(row0, ROWS_PER_TILE)],
                            acc.at[pl.ds(row0, ROWS_PER_TILE)])

        @pl.when(sid == NS - 1)
        def _():
            pltpu.sync_copy(x_hbm.at[pl.ds((NS - 1) * ROWS_PER_TILE, LAST_ROWS)],
                            acc.at[pl.ds((NS - 1) * ROWS_PER_TILE, LAST_ROWS)])

        plsc.subcore_barrier()

        pltpu.sync_copy(combo_hbm.at[pl.ds(wid * K, K)], combo_v)

        def unpack_idx(c, slot):
            for i in range(CHUNK // 16):
                w = combo_v[c, pl.ds(i * 16, 16)]
                srcs[slot][pl.ds(i * 16, 16)] = w & IDX_MASK
                dsts[slot][pl.ds(i * 16, 16)] = lax.shift_right_logical(
                    w, IDX_SHIFT)

        def gstart(b):
            pltpu.async_copy(x_hbm.at[srcs[b]], bufs[b], gsems[b])

        def gwait(b):
            pltpu.make_async_copy(x_hbm.at[srcs[b]], bufs[b], gsems[b]).wait()

        def body(m, carry):
            t0 = m * 2
            unpack_idx(t0, 0)
            gstart(0)
            unpack_idx(t0 + 1, 1)
            gstart(1)
            gwait(0)
            pltpu.sync_copy(bufs[0], acc.at[dsts[0]], add=True)
            gwait(1)
            pltpu.sync_copy(bufs[1], acc.at[dsts[1]], add=True)
            return carry

        lax.fori_loop(0, K // 2, body, 0)

        plsc.subcore_barrier()

        pltpu.sync_copy(acc.at[pl.ds(row0, ROWS_PER_TILE)],
                        out_hbm.at[cid, pl.ds(row0, ROWS_PER_TILE)])

    return agg_kernel(x, combo2)


BR = 1000


def _mlp_body(p_ref, x_ref, w1_ref, c1_ref, w2_ref, c2_ref, w3_ref, c3_ref, o_ref):
    h = p_ref[0] + p_ref[1] - x_ref[...]
    h = jnp.maximum(jnp.dot(h, w1_ref[...], preferred_element_type=jnp.float32)
                    + c1_ref[...], 0.0)
    h = jnp.maximum(jnp.dot(h, w2_ref[...], preferred_element_type=jnp.float32)
                    + c2_ref[...], 0.0)
    o_ref[...] = (jnp.dot(h, w3_ref[...], preferred_element_type=jnp.float32)
                  + c3_ref[...])


def _tc_mlp(parts, x, w1, c1, w2, c2, w3, c3):
    grid = N_NODES // BR
    return pl.pallas_call(
        _mlp_body,
        grid=(grid,),
        in_specs=[
            pl.BlockSpec((NC, BR, D), lambda i: (0, i, 0)),
            pl.BlockSpec((BR, D), lambda i: (i, 0)),
            pl.BlockSpec((D, D), lambda i: (0, 0)),
            pl.BlockSpec((1, D), lambda i: (0, 0)),
            pl.BlockSpec((D, D), lambda i: (0, 0)),
            pl.BlockSpec((1, D), lambda i: (0, 0)),
            pl.BlockSpec((D, D), lambda i: (0, 0)),
            pl.BlockSpec((1, D), lambda i: (0, 0)),
        ],
        out_specs=pl.BlockSpec((BR, D), lambda i: (i, 0)),
        out_shape=jax.ShapeDtypeStruct((N_NODES, D), jnp.float32),
    )(parts, x, w1, c1, w2, c2, w3, c3)


def kernel(x, edge_index, W1, b1, g1, be1, m1, v1, W2, b2, g2, be2, m2, v2, W3, b3):
    src = edge_index[0].astype(jnp.int32)
    dst = edge_index[1].astype(jnp.int32)
    combo = jnp.bitwise_or(jnp.left_shift(dst, IDX_SHIFT), src)
    pad = E_PAD - N_EDGES
    pad_dst = N_NODES + jnp.arange(pad, dtype=jnp.int32) % (N_PAD - N_NODES)
    combo_p = jnp.concatenate(
        [combo, jnp.left_shift(pad_dst, IDX_SHIFT)])
    combo2 = combo_p.reshape(NW * K, CHUNK)

    parts = _sc_aggregate(x, combo2)

    s1 = g1 * lax.rsqrt(v1 + BN_EPS)
    w1 = W1 * s1[None, :]
    c1 = (b1 * s1 + be1 - m1 * s1)[None, :]
    s2 = g2 * lax.rsqrt(v2 + BN_EPS)
    w2 = W2 * s2[None, :]
    c2 = (b2 * s2 + be2 - m2 * s2)[None, :]
    c3 = b3[None, :]

    return _tc_mlp(parts, x, w1, c1, w2, c2, W3, c3)


# speedup vs baseline: 3.6599x; 1.0524x over previous
import functools

import jax
import jax.numpy as jnp
from jax import lax
from jax.experimental import pallas as pl
from jax.experimental.pallas import tpu as pltpu
from jax.experimental.pallas import tpu_sc as plsc

N_NODES = 10000
D = 128
N_EDGES = 320000
BN_EPS = 1e-5

NC = 2
NS = 16
NW = NC * NS

CHUNK = 128
K = 80
E_PAD = NW * K * CHUNK
ROWS_PER_TILE = 632
LAST_ROWS = N_NODES - 15 * ROWS_PER_TILE
N_PAD = NS * ROWS_PER_TILE
IDX_SHIFT = 14
IDX_MASK = (1 << IDX_SHIFT) - 1


def _sc_aggregate(x, combo2):
    mesh = plsc.VectorSubcoreMesh(core_axis_name="c", subcore_axis_name="s")

    @functools.partial(
        pl.kernel,
        mesh=mesh,
        out_type=jax.ShapeDtypeStruct((NC, N_PAD, D), jnp.float32),
        scratch_types=[
            pltpu.VMEM((K, CHUNK), jnp.int32),
            pltpu.VMEM((CHUNK,), jnp.int32),
            pltpu.VMEM((CHUNK,), jnp.int32),
            pltpu.VMEM((CHUNK,), jnp.int32),
            pltpu.VMEM((CHUNK,), jnp.int32),
            pltpu.VMEM((CHUNK, D), jnp.float32),
            pltpu.VMEM((CHUNK, D), jnp.float32),
            pltpu.VMEM_SHARED((N_PAD, D), jnp.float32),
            pltpu.SemaphoreType.DMA,
            pltpu.SemaphoreType.DMA,
        ],
    )
    def agg_kernel(x_hbm, combo_hbm, out_hbm, combo_v, s0, s1, d0, d1,
                   b0, b1, acc, g0, g1):
        bufs = (b0, b1)
        srcs = (s0, s1)
        dsts = (d0, d1)
        gsems = (g0, g1)
        cid = lax.axis_index("c")
        sid = lax.axis_index("s")
        wid = (1 - cid) * NS + sid

        row0 = sid * ROWS_PER_TILE

        @pl.when(sid < NS - 1)
        def _():
            pltpu.sync_copy(x_hbm.at[pl.ds(row0, ROWS_PER_TILE)],
                            acc.at[pl.ds(row0, ROWS_PER_TILE)])

        @pl.when(sid == NS - 1)
        def _():
            pltpu.sync_copy(x_hbm.at[pl.ds((NS - 1) * ROWS_PER_TILE, LAST_ROWS)],
                            acc.at[pl.ds((NS - 1) * ROWS_PER_TILE, LAST_ROWS)])

        plsc.subcore_barrier()

        pltpu.sync_copy(combo_hbm.at[pl.ds(wid * K, K)], combo_v)

        def unpack_idx(c, slot):
            for i in range(CHUNK // 16):
                w = combo_v[c, pl.ds(i * 16, 16)]
                srcs[slot][pl.ds(i * 16, 16)] = w & IDX_MASK
                dsts[slot][pl.ds(i * 16, 16)] = lax.shift_right_logical(
                    w, IDX_SHIFT)

        def gstart(b):
            pltpu.async_copy(x_hbm.at[srcs[b]], bufs[b], gsems[b])

        def gwait(b):
            pltpu.make_async_copy(x_hbm.at[srcs[b]], bufs[b], gsems[b]).wait()

        def body(m, carry):
            t0 = m * 2
            unpack_idx(t0, 0)
            gstart(0)
            unpack_idx(t0 + 1, 1)
            gstart(1)
            gwait(0)
            pltpu.sync_copy(bufs[0], acc.at[dsts[0]], add=True)
            gwait(1)
            pltpu.sync_copy(bufs[1], acc.at[dsts[1]], add=True)
            return carry

        lax.fori_loop(0, K // 2, body, 0)

        plsc.subcore_barrier()

        pltpu.sync_copy(acc.at[pl.ds(row0, ROWS_PER_TILE)],
                        out_hbm.at[cid, pl.ds(row0, ROWS_PER_TILE)])

    return agg_kernel(x, combo2)


BR = 1000


def _mlp_body(p_ref, x_ref, w1_ref, c1_ref, w2_ref, c2_ref, w3_ref, c3_ref, o_ref):
    h = p_ref[0] + p_ref[1] - x_ref[...]
    h = jnp.maximum(jnp.dot(h, w1_ref[...], preferred_element_type=jnp.float32)
                    + c1_ref[...], 0.0)
    h = jnp.maximum(jnp.dot(h, w2_ref[...], preferred_element_type=jnp.float32)
                    + c2_ref[...], 0.0)
    o_ref[...] = (jnp.dot(h, w3_ref[...], preferred_element_type=jnp.float32)
                  + c3_ref[...])


def _tc_mlp(parts, x, w1, c1, w2, c2, w3, c3):
    grid = N_NODES // BR
    return pl.pallas_call(
        _mlp_body,
        grid=(grid,),
        in_specs=[
            pl.BlockSpec((NC, BR, D), lambda i: (0, i, 0)),
            pl.BlockSpec((BR, D), lambda i: (i, 0)),
            pl.BlockSpec((D, D), lambda i: (0, 0)),
            pl.BlockSpec((1, D), lambda i: (0, 0)),
            pl.BlockSpec((D, D), lambda i: (0, 0)),
            pl.BlockSpec((1, D), lambda i: (0, 0)),
            pl.BlockSpec((D, D), lambda i: (0, 0)),
            pl.BlockSpec((1, D), lambda i: (0, 0)),
        ],
        out_specs=pl.BlockSpec((BR, D), lambda i: (i, 0)),
        out_shape=jax.ShapeDtypeStruct((N_NODES, D), jnp.float32),
    )(parts, x, w1, c1, w2, c2, w3, c3)


def kernel(x, edge_index, W1, b1, g1, be1, m1, v1, W2, b2, g2, be2, m2, v2, W3, b3):
    src = edge_index[0].astype(jnp.int32)
    dst = edge_index[1].astype(jnp.int32)
    combo = jnp.bitwise_or(jnp.left_shift(dst, IDX_SHIFT), src)
    pad = E_PAD - N_EDGES
    pad_dst = N_NODES + jnp.arange(pad, dtype=jnp.int32) % (N_PAD - N_NODES)
    combo_p = jnp.concatenate(
        [combo, jnp.left_shift(pad_dst, IDX_SHIFT)])
    combo2 = combo_p.reshape(NW * K, CHUNK)

    parts = _sc_aggregate(x, combo2)

    s1 = g1 * lax.rsqrt(v1 + BN_EPS)
    w1 = W1 * s1[None, :]
    c1 = (b1 * s1 + be1 - m1 * s1)[None, :]
    s2 = g2 * lax.rsqrt(v2 + BN_EPS)
    w2 = W2 * s2[None, :]
    c2 = (b2 * s2 + be2 - m2 * s2)[None, :]
    c3 = b3[None, :]

    return _tc_mlp(parts, x, w1, c1, w2, c2, W3, c3)


# speedup vs baseline: 9.8153x; 2.6819x over previous
import functools

import jax
import jax.numpy as jnp
from jax import lax
from jax.experimental import pallas as pl
from jax.experimental.pallas import tpu as pltpu
from jax.experimental.pallas import tpu_sc as plsc

N_NODES = 10000
D = 128
N_EDGES = 320000
BN_EPS = 1e-5

NC = 2
NS = 16
NW = NC * NS

CHUNK = 128
K = 80
E_PAD = NW * K * CHUNK
ROWS_PER_TILE = 632
LAST_ROWS = N_NODES - 15 * ROWS_PER_TILE
N_PAD = NS * ROWS_PER_TILE
IDX_SHIFT = 14
IDX_MASK = (1 << IDX_SHIFT) - 1


def _sc_aggregate(x, combo2):
    mesh = plsc.VectorSubcoreMesh(core_axis_name="c", subcore_axis_name="s")

    @functools.partial(
        pl.kernel,
        mesh=mesh,
        out_type=jax.ShapeDtypeStruct((NC, N_PAD, D), jnp.float32),
        scratch_types=[
            pltpu.VMEM((K, CHUNK), jnp.int32),
            pltpu.VMEM((CHUNK,), jnp.int32),
            pltpu.VMEM((CHUNK,), jnp.int32),
            pltpu.VMEM((CHUNK,), jnp.int32),
            pltpu.VMEM((CHUNK,), jnp.int32),
            pltpu.VMEM((CHUNK, D), jnp.float32),
            pltpu.VMEM((CHUNK, D), jnp.float32),
            pltpu.VMEM_SHARED((N_PAD, D), jnp.float32),
            pltpu.SemaphoreType.DMA,
            pltpu.SemaphoreType.DMA,
        ],
    )
    def agg_kernel(x_hbm, combo_hbm, out_hbm, combo_v, s0, s1, d0, d1,
                   b0, b1, acc, g0, g1):
        bufs = (b0, b1)
        srcs = (s0, s1)
        dsts = (d0, d1)
        gsems = (g0, g1)
        cid = lax.axis_index("c")
        sid = lax.axis_index("s")
        wid = cid * NS + sid

        row0 = sid * ROWS_PER_TILE

        @pl.when(sid < NS - 1)
        def _():
            pltpu.sync_copy(x_hbm.at[pl.ds(row0, ROWS_PER_TILE)],
                            acc.at[pl.ds(row0, ROWS_PER_TILE)])

        @pl.when(sid == NS - 1)
        def _():
            pltpu.sync_copy(x_hbm.at[pl.ds((NS - 1) * ROWS_PER_TILE, LAST_ROWS)],
                            acc.at[pl.ds((NS - 1) * ROWS_PER_TILE, LAST_ROWS)])

        plsc.subcore_barrier()

        pltpu.sync_copy(combo_hbm.at[pl.ds(wid * K, K)], combo_v)

        def unpack_idx(c, slot):
            for i in range(CHUNK // 16):
                w = combo_v[c, pl.ds(i * 16, 16)]
                srcs[slot][pl.ds(i * 16, 16)] = w & IDX_MASK
                dsts[slot][pl.ds(i * 16, 16)] = lax.shift_right_logical(
                    w, IDX_SHIFT)

        def gstart(b):
            pltpu.async_copy(x_hbm.at[srcs[b]], bufs[b], gsems[b])

        def gwait(b):
            pltpu.make_async_copy(x_hbm.at[srcs[b]], bufs[b], gsems[b]).wait()

        def body(m, carry):
            t0 = m * 2
            unpack_idx(t0, 0)
            gstart(0)
            unpack_idx(t0 + 1, 1)
            gstart(1)
            gwait(0)
            pltpu.sync_copy(bufs[0], acc.at[dsts[0]], add=True)
            gwait(1)
            pltpu.sync_copy(bufs[1], acc.at[dsts[1]], add=True)
            return carry

        lax.fori_loop(0, K // 2, body, 0)

        plsc.subcore_barrier()

        pltpu.sync_copy(acc.at[pl.ds(row0, ROWS_PER_TILE)],
                        out_hbm.at[cid, pl.ds(row0, ROWS_PER_TILE)])

    return agg_kernel(x, combo2)


BR = 1000


def _mlp_body(p_ref, x_ref, w1_ref, c1_ref, w2_ref, c2_ref, w3_ref, c3_ref, o_ref):
    h = p_ref[0] + p_ref[1] - x_ref[...]
    h = jnp.maximum(jnp.dot(h, w1_ref[...], preferred_element_type=jnp.float32)
                    + c1_ref[...], 0.0)
    h = jnp.maximum(jnp.dot(h, w2_ref[...], preferred_element_type=jnp.float32)
                    + c2_ref[...], 0.0)
    o_ref[...] = (jnp.dot(h, w3_ref[...], preferred_element_type=jnp.float32)
                  + c3_ref[...])


def _tc_mlp(parts, x, w1, c1, w2, c2, w3, c3):
    grid = N_NODES // BR
    return pl.pallas_call(
        _mlp_body,
        grid=(grid,),
        in_specs=[
            pl.BlockSpec((NC, BR, D), lambda i: (0, i, 0)),
            pl.BlockSpec((BR, D), lambda i: (i, 0)),
            pl.BlockSpec((D, D), lambda i: (0, 0)),
            pl.BlockSpec((1, D), lambda i: (0, 0)),
            pl.BlockSpec((D, D), lambda i: (0, 0)),
            pl.BlockSpec((1, D), lambda i: (0, 0)),
            pl.BlockSpec((D, D), lambda i: (0, 0)),
            pl.BlockSpec((1, D), lambda i: (0, 0)),
        ],
        out_specs=pl.BlockSpec((BR, D), lambda i: (i, 0)),
        out_shape=jax.ShapeDtypeStruct((N_NODES, D), jnp.float32),
    )(parts, x, w1, c1, w2, c2, w3, c3)


def kernel(x, edge_index, W1, b1, g1, be1, m1, v1, W2, b2, g2, be2, m2, v2, W3, b3):
    src = edge_index[0].astype(jnp.int32)
    dst = edge_index[1].astype(jnp.int32)
    combo = jnp.bitwise_or(jnp.left_shift(dst, IDX_SHIFT), src)
    pad = E_PAD - N_EDGES
    pad_i = jnp.arange(pad, dtype=jnp.int32)
    pad_dst = N_NODES + pad_i % (N_PAD - N_NODES)
    pad_src = (pad_i * 13) % N_NODES
    combo_p = jnp.concatenate(
        [combo, jnp.bitwise_or(jnp.left_shift(pad_dst, IDX_SHIFT), pad_src)])
    combo2 = combo_p.reshape(NW * K, CHUNK)

    parts = _sc_aggregate(x, combo2)

    s1 = g1 * lax.rsqrt(v1 + BN_EPS)
    w1 = W1 * s1[None, :]
    c1 = (b1 * s1 + be1 - m1 * s1)[None, :]
    s2 = g2 * lax.rsqrt(v2 + BN_EPS)
    w2 = W2 * s2[None, :]
    c2 = (b2 * s2 + be2 - m2 * s2)[None, :]
    c3 = b3[None, :]

    return _tc_mlp(parts, x, w1, c1, w2, c2, W3, c3)


# speedup vs baseline: 12.6276x; 1.2865x over previous
import functools

import jax
import jax.numpy as jnp
from jax import lax
from jax.experimental import pallas as pl
from jax.experimental.pallas import tpu as pltpu
from jax.experimental.pallas import tpu_sc as plsc

N_NODES = 10000
D = 128
N_EDGES = 320000
BN_EPS = 1e-5

NC = 2
NS = 16
NW = NC * NS

CHUNK = 128
K = 80
E_PAD = NW * K * CHUNK
ROWS_PER_TILE = 632
LAST_ROWS = N_NODES - 15 * ROWS_PER_TILE
N_PAD = NS * ROWS_PER_TILE
IDX_SHIFT = 14
IDX_MASK = (1 << IDX_SHIFT) - 1


def _sc_aggregate(x, combo2):
    mesh = plsc.VectorSubcoreMesh(core_axis_name="c", subcore_axis_name="s")

    @functools.partial(
        pl.kernel,
        mesh=mesh,
        out_type=jax.ShapeDtypeStruct((NC, N_PAD, D), jnp.float32),
        scratch_types=[
            pltpu.VMEM((K, CHUNK), jnp.int32),
            pltpu.VMEM((CHUNK,), jnp.int32),
            pltpu.VMEM((CHUNK,), jnp.int32),
            pltpu.VMEM((CHUNK,), jnp.int32),
            pltpu.VMEM((CHUNK,), jnp.int32),
            pltpu.VMEM((CHUNK, D), jnp.float32),
            pltpu.VMEM((CHUNK, D), jnp.float32),
            pltpu.VMEM_SHARED((N_PAD, D), jnp.float32),
            pltpu.SemaphoreType.DMA,
            pltpu.SemaphoreType.DMA,
            pltpu.SemaphoreType.DMA,
            pltpu.SemaphoreType.DMA,
        ],
    )
    def agg_kernel(x_hbm, combo_hbm, out_hbm, combo_v, s0, s1, d0, d1,
                   b0, b1, acc, g0, g1, sc0, sc1):
        bufs = (b0, b1)
        srcs = (s0, s1)
        dsts = (d0, d1)
        gsems = (g0, g1)
        cid = lax.axis_index("c")
        sid = lax.axis_index("s")
        wid = cid * NS + sid

        row0 = sid * ROWS_PER_TILE

        @pl.when(sid < NS - 1)
        def _():
            pltpu.sync_copy(x_hbm.at[pl.ds(row0, ROWS_PER_TILE)],
                            acc.at[pl.ds(row0, ROWS_PER_TILE)])

        @pl.when(sid == NS - 1)
        def _():
            pltpu.sync_copy(x_hbm.at[pl.ds((NS - 1) * ROWS_PER_TILE, LAST_ROWS)],
                            acc.at[pl.ds((NS - 1) * ROWS_PER_TILE, LAST_ROWS)])

        plsc.subcore_barrier()

        pltpu.sync_copy(combo_hbm.at[pl.ds(wid * K, K)], combo_v)

        def unpack_idx(c, slot):
            for i in range(CHUNK // 16):
                w = combo_v[c, pl.ds(i * 16, 16)]
                srcs[slot][pl.ds(i * 16, 16)] = w & IDX_MASK
                dsts[slot][pl.ds(i * 16, 16)] = lax.shift_right_logical(
                    w, IDX_SHIFT)

        ssems = (sc0, sc1)

        def gstart(b):
            pltpu.async_copy(x_hbm.at[srcs[b]], bufs[b], gsems[b])

        def gwait(b):
            pltpu.make_async_copy(x_hbm.at[srcs[b]], bufs[b], gsems[b]).wait()

        def sstart(b):
            pltpu.async_copy(bufs[b], acc.at[dsts[b]], ssems[b], add=True)

        def swait(b):
            pltpu.make_async_copy(bufs[b], acc.at[dsts[b]], ssems[b]).wait()

        unpack_idx(0, 0)
        gstart(0)
        unpack_idx(1, 1)
        gstart(1)
        gwait(0)
        sstart(0)

        def step(t, b):
            swait(b)
            unpack_idx(t, b)
            gstart(b)
            gwait(1 - b)
            sstart(1 - b)

        def body(m, carry):
            step(2 * m + 2, 0)
            step(2 * m + 3, 1)
            return carry

        lax.fori_loop(0, (K - 2) // 2, body, 0)

        gwait(1)
        sstart(1)
        swait(0)
        swait(1)

        plsc.subcore_barrier()

        pltpu.sync_copy(acc.at[pl.ds(row0, ROWS_PER_TILE)],
                        out_hbm.at[cid, pl.ds(row0, ROWS_PER_TILE)])

    return agg_kernel(x, combo2)


BR = 1000


def _mlp_body(p_ref, x_ref, w1_ref, c1_ref, w2_ref, c2_ref, w3_ref, c3_ref, o_ref):
    h = p_ref[0] + p_ref[1] - x_ref[...]
    h = jnp.maximum(jnp.dot(h, w1_ref[...], preferred_element_type=jnp.float32)
                    + c1_ref[...], 0.0)
    h = jnp.maximum(jnp.dot(h, w2_ref[...], preferred_element_type=jnp.float32)
                    + c2_ref[...], 0.0)
    o_ref[...] = (jnp.dot(h, w3_ref[...], preferred_element_type=jnp.float32)
                  + c3_ref[...])


def _tc_mlp(parts, x, w1, c1, w2, c2, w3, c3):
    grid = N_NODES // BR
    return pl.pallas_call(
        _mlp_body,
        grid=(grid,),
        in_specs=[
            pl.BlockSpec((NC, BR, D), lambda i: (0, i, 0)),
            pl.BlockSpec((BR, D), lambda i: (i, 0)),
            pl.BlockSpec((D, D), lambda i: (0, 0)),
            pl.BlockSpec((1, D), lambda i: (0, 0)),
            pl.BlockSpec((D, D), lambda i: (0, 0)),
            pl.BlockSpec((1, D), lambda i: (0, 0)),
            pl.BlockSpec((D, D), lambda i: (0, 0)),
            pl.BlockSpec((1, D), lambda i: (0, 0)),
        ],
        out_specs=pl.BlockSpec((BR, D), lambda i: (i, 0)),
        out_shape=jax.ShapeDtypeStruct((N_NODES, D), jnp.float32),
    )(parts, x, w1, c1, w2, c2, w3, c3)


def kernel(x, edge_index, W1, b1, g1, be1, m1, v1, W2, b2, g2, be2, m2, v2, W3, b3):
    src = edge_index[0].astype(jnp.int32)
    dst = edge_index[1].astype(jnp.int32)
    combo = jnp.bitwise_or(jnp.left_shift(dst, IDX_SHIFT), src)
    pad = E_PAD - N_EDGES
    pad_i = jnp.arange(pad, dtype=jnp.int32)
    pad_dst = N_NODES + pad_i % (N_PAD - N_NODES)
    pad_src = (pad_i * 13) % N_NODES
    combo_p = jnp.concatenate(
        [combo, jnp.bitwise_or(jnp.left_shift(pad_dst, IDX_SHIFT), pad_src)])
    combo2 = combo_p.reshape(NW * K, CHUNK)

    parts = _sc_aggregate(x, combo2)

    s1 = g1 * lax.rsqrt(v1 + BN_EPS)
    w1 = W1 * s1[None, :]
    c1 = (b1 * s1 + be1 - m1 * s1)[None, :]
    s2 = g2 * lax.rsqrt(v2 + BN_EPS)
    w2 = W2 * s2[None, :]
    c2 = (b2 * s2 + be2 - m2 * s2)[None, :]
    c3 = b3[None, :]

    return _tc_mlp(parts, x, w1, c1, w2, c2, W3, c3)


# speedup vs baseline: 12.6312x; 1.0003x over previous
import functools

import jax
import jax.numpy as jnp
from jax import lax
from jax.experimental import pallas as pl
from jax.experimental.pallas import tpu as pltpu
from jax.experimental.pallas import tpu_sc as plsc

N_NODES = 10000
D = 128
N_EDGES = 320000
BN_EPS = 1e-5

NC = 2
NS = 16
NW = NC * NS

CHUNK = 128
K = 80
E_PAD = NW * K * CHUNK
ROWS_PER_TILE = 632
LAST_ROWS = N_NODES - 15 * ROWS_PER_TILE
N_PAD = NS * ROWS_PER_TILE
IDX_SHIFT = 14
IDX_MASK = (1 << IDX_SHIFT) - 1


def _sc_aggregate(x, combo2):
    mesh = plsc.VectorSubcoreMesh(core_axis_name="c", subcore_axis_name="s")

    @functools.partial(
        pl.kernel,
        mesh=mesh,
        out_type=jax.ShapeDtypeStruct((NC, N_PAD, D), jnp.float32),
        scratch_types=[
            pltpu.VMEM((K, CHUNK), jnp.int32),
            pltpu.VMEM((CHUNK,), jnp.int32),
            pltpu.VMEM((CHUNK,), jnp.int32),
            pltpu.VMEM((CHUNK // 2,), jnp.int32),
            pltpu.VMEM((CHUNK // 2,), jnp.int32),
            pltpu.VMEM((CHUNK // 2,), jnp.int32),
            pltpu.VMEM((CHUNK // 2,), jnp.int32),
            pltpu.VMEM((CHUNK, D), jnp.float32),
            pltpu.VMEM((CHUNK, D), jnp.float32),
            pltpu.VMEM_SHARED((N_PAD, D), jnp.float32),
            pltpu.SemaphoreType.DMA,
            pltpu.SemaphoreType.DMA,
            pltpu.SemaphoreType.DMA,
            pltpu.SemaphoreType.DMA,
            pltpu.SemaphoreType.DMA,
            pltpu.SemaphoreType.DMA,
        ],
    )
    def agg_kernel(x_hbm, combo_hbm, out_hbm, combo_v, s0, s1,
                   d0a, d0b, d1a, d1b, b0, b1, acc,
                   g0, g1, sa0, sb0, sa1, sb1):
        bufs = (b0, b1)
        srcs = (s0, s1)
        dsts_lo = (d0a, d1a)
        dsts_hi = (d0b, d1b)
        gsems = (g0, g1)
        ssems_lo = (sa0, sa1)
        ssems_hi = (sb0, sb1)
        cid = lax.axis_index("c")
        sid = lax.axis_index("s")
        wid = cid * NS + sid

        row0 = sid * ROWS_PER_TILE

        @pl.when(sid < NS - 1)
        def _():
            pltpu.sync_copy(x_hbm.at[pl.ds(row0, ROWS_PER_TILE)],
                            acc.at[pl.ds(row0, ROWS_PER_TILE)])

        @pl.when(sid == NS - 1)
        def _():
            pltpu.sync_copy(x_hbm.at[pl.ds((NS - 1) * ROWS_PER_TILE, LAST_ROWS)],
                            acc.at[pl.ds((NS - 1) * ROWS_PER_TILE, LAST_ROWS)])

        plsc.subcore_barrier()

        pltpu.sync_copy(combo_hbm.at[pl.ds(wid * K, K)], combo_v)

        def unpack_idx(c, slot):
            half = CHUNK // 32
            for i in range(CHUNK // 16):
                w = combo_v[c, pl.ds(i * 16, 16)]
                srcs[slot][pl.ds(i * 16, 16)] = w & IDX_MASK
                d = lax.shift_right_logical(w, IDX_SHIFT)
                if i < half:
                    dsts_lo[slot][pl.ds(i * 16, 16)] = d
                else:
                    dsts_hi[slot][pl.ds((i - half) * 16, 16)] = d

        H = CHUNK // 2

        def gstart(b):
            pltpu.async_copy(x_hbm.at[srcs[b]], bufs[b], gsems[b])

        def gwait(b):
            pltpu.make_async_copy(x_hbm.at[srcs[b]], bufs[b], gsems[b]).wait()

        def sstart(b):
            pltpu.async_copy(bufs[b].at[pl.ds(0, H)], acc.at[dsts_lo[b]],
                             ssems_lo[b], add=True)
            pltpu.async_copy(bufs[b].at[pl.ds(H, H)], acc.at[dsts_hi[b]],
                             ssems_hi[b], add=True)

        def swait(b):
            pltpu.make_async_copy(bufs[b].at[pl.ds(0, H)], acc.at[dsts_lo[b]],
                                  ssems_lo[b]).wait()
            pltpu.make_async_copy(bufs[b].at[pl.ds(H, H)], acc.at[dsts_hi[b]],
                                  ssems_hi[b]).wait()

        unpack_idx(0, 0)
        gstart(0)
        unpack_idx(1, 1)
        gstart(1)
        gwait(0)
        sstart(0)

        def step(t, b):
            swait(b)
            unpack_idx(t, b)
            gstart(b)
            gwait(1 - b)
            sstart(1 - b)

        def body(m, carry):
            step(2 * m + 2, 0)
            step(2 * m + 3, 1)
            return carry

        lax.fori_loop(0, (K - 2) // 2, body, 0)

        gwait(1)
        sstart(1)
        swait(0)
        swait(1)

        plsc.subcore_barrier()

        pltpu.sync_copy(acc.at[pl.ds(row0, ROWS_PER_TILE)],
                        out_hbm.at[cid, pl.ds(row0, ROWS_PER_TILE)])

    return agg_kernel(x, combo2)


BR = 1000


def _mlp_body(p_ref, x_ref, w1_ref, c1_ref, w2_ref, c2_ref, w3_ref, c3_ref, o_ref):
    h = p_ref[0] + p_ref[1] - x_ref[...]
    h = jnp.maximum(jnp.dot(h, w1_ref[...], preferred_element_type=jnp.float32)
                    + c1_ref[...], 0.0)
    h = jnp.maximum(jnp.dot(h, w2_ref[...], preferred_element_type=jnp.float32)
                    + c2_ref[...], 0.0)
    o_ref[...] = (jnp.dot(h, w3_ref[...], preferred_element_type=jnp.float32)
                  + c3_ref[...])


def _tc_mlp(parts, x, w1, c1, w2, c2, w3, c3):
    grid = N_NODES // BR
    return pl.pallas_call(
        _mlp_body,
        grid=(grid,),
        in_specs=[
            pl.BlockSpec((NC, BR, D), lambda i: (0, i, 0)),
            pl.BlockSpec((BR, D), lambda i: (i, 0)),
            pl.BlockSpec((D, D), lambda i: (0, 0)),
            pl.BlockSpec((1, D), lambda i: (0, 0)),
            pl.BlockSpec((D, D), lambda i: (0, 0)),
            pl.BlockSpec((1, D), lambda i: (0, 0)),
            pl.BlockSpec((D, D), lambda i: (0, 0)),
            pl.BlockSpec((1, D), lambda i: (0, 0)),
        ],
        out_specs=pl.BlockSpec((BR, D), lambda i: (i, 0)),
        out_shape=jax.ShapeDtypeStruct((N_NODES, D), jnp.float32),
    )(parts, x, w1, c1, w2, c2, w3, c3)


def kernel(x, edge_index, W1, b1, g1, be1, m1, v1, W2, b2, g2, be2, m2, v2, W3, b3):
    src = edge_index[0].astype(jnp.int32)
    dst = edge_index[1].astype(jnp.int32)
    combo = jnp.bitwise_or(jnp.left_shift(dst, IDX_SHIFT), src)
    pad = E_PAD - N_EDGES
    pad_i = jnp.arange(pad, dtype=jnp.int32)
    pad_dst = N_NODES + pad_i % (N_PAD - N_NODES)
    pad_src = (pad_i * 13) % N_NODES
    combo_p = jnp.concatenate(
        [combo, jnp.bitwise_or(jnp.left_shift(pad_dst, IDX_SHIFT), pad_src)])
    combo2 = combo_p.reshape(NW * K, CHUNK)

    parts = _sc_aggregate(x, combo2)

    s1 = g1 * lax.rsqrt(v1 + BN_EPS)
    w1 = W1 * s1[None, :]
    c1 = (b1 * s1 + be1 - m1 * s1)[None, :]
    s2 = g2 * lax.rsqrt(v2 + BN_EPS)
    w2 = W2 * s2[None, :]
    c2 = (b2 * s2 + be2 - m2 * s2)[None, :]
    c3 = b3[None, :]

    return _tc_mlp(parts, x, w1, c1, w2, c2, W3, c3)


# speedup vs baseline: 12.9260x; 1.0233x over previous
import functools

import jax
import jax.numpy as jnp
import numpy as np
from jax import lax
from jax.experimental import pallas as pl
from jax.experimental.pallas import tpu as pltpu
from jax.experimental.pallas import tpu_sc as plsc

N_NODES = 10000
D = 128
N_EDGES = 320000
BN_EPS = 1e-5

NC = 2
NS = 16
NW = NC * NS

CHUNK = 128
K = 80
E_PAD = NW * K * CHUNK
ROWS_PER_TILE = 632
LAST_ROWS = N_NODES - 15 * ROWS_PER_TILE
N_PAD = NS * ROWS_PER_TILE
IDX_SHIFT = 14
IDX_MASK = (1 << IDX_SHIFT) - 1


def _sc_aggregate(x, combo2):
    mesh = plsc.VectorSubcoreMesh(core_axis_name="c", subcore_axis_name="s")

    @functools.partial(
        pl.kernel,
        mesh=mesh,
        out_type=jax.ShapeDtypeStruct((NC, N_PAD, D), jnp.float32),
        scratch_types=[
            pltpu.VMEM((K, CHUNK), jnp.int32),
            pltpu.VMEM((CHUNK,), jnp.int32),
            pltpu.VMEM((CHUNK,), jnp.int32),
            pltpu.VMEM((CHUNK // 2,), jnp.int32),
            pltpu.VMEM((CHUNK // 2,), jnp.int32),
            pltpu.VMEM((CHUNK // 2,), jnp.int32),
            pltpu.VMEM((CHUNK // 2,), jnp.int32),
            pltpu.VMEM((CHUNK, D), jnp.float32),
            pltpu.VMEM((CHUNK, D), jnp.float32),
            pltpu.VMEM_SHARED((N_PAD, D), jnp.float32),
            pltpu.SemaphoreType.DMA,
            pltpu.SemaphoreType.DMA,
            pltpu.SemaphoreType.DMA,
            pltpu.SemaphoreType.DMA,
            pltpu.SemaphoreType.DMA,
            pltpu.SemaphoreType.DMA,
        ],
    )
    def agg_kernel(x_hbm, combo_hbm, out_hbm, combo_v, s0, s1,
                   d0a, d0b, d1a, d1b, b0, b1, acc,
                   g0, g1, sa0, sb0, sa1, sb1):
        bufs = (b0, b1)
        srcs = (s0, s1)
        dsts_lo = (d0a, d1a)
        dsts_hi = (d0b, d1b)
        gsems = (g0, g1)
        ssems_lo = (sa0, sa1)
        ssems_hi = (sb0, sb1)
        cid = lax.axis_index("c")
        sid = lax.axis_index("s")
        wid = cid * NS + sid

        row0 = sid * ROWS_PER_TILE

        @pl.when(sid < NS - 1)
        def _():
            pltpu.sync_copy(x_hbm.at[pl.ds(row0, ROWS_PER_TILE)],
                            acc.at[pl.ds(row0, ROWS_PER_TILE)])

        @pl.when(sid == NS - 1)
        def _():
            pltpu.sync_copy(x_hbm.at[pl.ds((NS - 1) * ROWS_PER_TILE, LAST_ROWS)],
                            acc.at[pl.ds((NS - 1) * ROWS_PER_TILE, LAST_ROWS)])

        plsc.subcore_barrier()

        pltpu.sync_copy(combo_hbm.at[pl.ds(wid * K, K)], combo_v)

        def unpack_idx(c, slot):
            half = CHUNK // 32
            for i in range(CHUNK // 16):
                w = combo_v[c, pl.ds(i * 16, 16)]
                srcs[slot][pl.ds(i * 16, 16)] = w & IDX_MASK
                d = lax.shift_right_logical(w, IDX_SHIFT)
                if i < half:
                    dsts_lo[slot][pl.ds(i * 16, 16)] = d
                else:
                    dsts_hi[slot][pl.ds((i - half) * 16, 16)] = d

        H = CHUNK // 2

        def gstart(b):
            pltpu.async_copy(x_hbm.at[srcs[b]], bufs[b], gsems[b])

        def gwait(b):
            pltpu.make_async_copy(x_hbm.at[srcs[b]], bufs[b], gsems[b]).wait()

        def sstart(b):
            pltpu.async_copy(bufs[b].at[pl.ds(0, H)], acc.at[dsts_lo[b]],
                             ssems_lo[b], add=True)
            pltpu.async_copy(bufs[b].at[pl.ds(H, H)], acc.at[dsts_hi[b]],
                             ssems_hi[b], add=True)

        def swait(b):
            pltpu.make_async_copy(bufs[b].at[pl.ds(0, H)], acc.at[dsts_lo[b]],
                                  ssems_lo[b]).wait()
            pltpu.make_async_copy(bufs[b].at[pl.ds(H, H)], acc.at[dsts_hi[b]],
                                  ssems_hi[b]).wait()

        unpack_idx(0, 0)
        gstart(0)
        unpack_idx(1, 1)
        gstart(1)
        gwait(0)
        sstart(0)

        def step(t, b):
            swait(b)
            unpack_idx(t, b)
            gstart(b)
            gwait(1 - b)
            sstart(1 - b)

        def body(m, carry):
            step(2 * m + 2, 0)
            step(2 * m + 3, 1)
            return carry

        lax.fori_loop(0, (K - 2) // 2, body, 0)

        gwait(1)
        sstart(1)
        swait(0)
        swait(1)

        plsc.subcore_barrier()

        pltpu.sync_copy(acc.at[pl.ds(row0, ROWS_PER_TILE)],
                        out_hbm.at[cid, pl.ds(row0, ROWS_PER_TILE)])

    return agg_kernel(x, combo2)


BR = 2000

_PAD_I = np.arange(E_PAD - N_EDGES)
_PAD_COMBO = np.asarray(
    ((N_NODES + _PAD_I % (N_PAD - N_NODES)) << IDX_SHIFT)
    | ((_PAD_I * 13) % N_NODES), dtype=np.int32)


def _mlp_body(p_ref, x_ref, w1_ref, b1_ref, g1_ref, be1_ref, m1_ref, v1_ref,
              w2_ref, b2_ref, g2_ref, be2_ref, m2_ref, v2_ref, w3_ref, b3_ref,
              o_ref):
    s1 = g1_ref[...] * lax.rsqrt(v1_ref[...] + BN_EPS)
    c1 = (b1_ref[...] - m1_ref[...]) * s1 + be1_ref[...]
    s2 = g2_ref[...] * lax.rsqrt(v2_ref[...] + BN_EPS)
    c2 = (b2_ref[...] - m2_ref[...]) * s2 + be2_ref[...]
    h = p_ref[0] + p_ref[1] - x_ref[...]
    h = jnp.maximum(jnp.dot(h, w1_ref[...] * s1,
                            preferred_element_type=jnp.float32) + c1, 0.0)
    h = jnp.maximum(jnp.dot(h, w2_ref[...] * s2,
                            preferred_element_type=jnp.float32) + c2, 0.0)
    o_ref[...] = (jnp.dot(h, w3_ref[...], preferred_element_type=jnp.float32)
                  + b3_ref[...])


def _tc_mlp(parts, x, *weights):
    grid = N_NODES // BR
    wspecs = []
    for w in weights:
        if w.ndim == 2:
            wspecs.append(pl.BlockSpec((D, D), lambda i: (0, 0)))
        else:
            wspecs.append(pl.BlockSpec((1, D), lambda i: (0, 0)))
    return pl.pallas_call(
        _mlp_body,
        grid=(grid,),
        in_specs=[
            pl.BlockSpec((NC, BR, D), lambda i: (0, i, 0)),
            pl.BlockSpec((BR, D), lambda i: (i, 0)),
            *wspecs,
        ],
        out_specs=pl.BlockSpec((BR, D), lambda i: (i, 0)),
        out_shape=jax.ShapeDtypeStruct((N_NODES, D), jnp.float32),
    )(parts, x, *(w.reshape(1, D) if w.ndim == 1 else w for w in weights))


def kernel(x, edge_index, W1, b1, g1, be1, m1, v1, W2, b2, g2, be2, m2, v2, W3, b3):
    src = edge_index[0].astype(jnp.int32)
    dst = edge_index[1].astype(jnp.int32)
    combo = jnp.bitwise_or(jnp.left_shift(dst, IDX_SHIFT), src)
    combo2 = jnp.concatenate([combo, jnp.asarray(_PAD_COMBO)]).reshape(
        NW * K, CHUNK)

    parts = _sc_aggregate(x, combo2)

    return _tc_mlp(parts, x, W1, b1, g1, be1, m1, v1,
                   W2, b2, g2, be2, m2, v2, W3, b3)


# speedup vs baseline: 13.5232x; 1.0462x over previous
import functools

import jax
import jax.numpy as jnp
import numpy as np
from jax import lax
from jax.experimental import pallas as pl
from jax.experimental.pallas import tpu as pltpu
from jax.experimental.pallas import tpu_sc as plsc

N_NODES = 10000
D = 128
N_EDGES = 320000
BN_EPS = 1e-5

NC = 2
NS = 16
NW = NC * NS

CHUNK = 128
K = 80
E_PAD = NW * K * CHUNK
ROWS_PER_TILE = 632
LAST_ROWS = N_NODES - 15 * ROWS_PER_TILE
N_PAD = NS * ROWS_PER_TILE
IDX_SHIFT = 14
IDX_MASK = (1 << IDX_SHIFT) - 1


def _sc_aggregate(x, combo2):
    mesh = plsc.VectorSubcoreMesh(core_axis_name="c", subcore_axis_name="s")

    @functools.partial(
        pl.kernel,
        mesh=mesh,
        out_type=jax.ShapeDtypeStruct((NC, N_PAD, D), jnp.float32),
        scratch_types=[
            pltpu.VMEM((K, CHUNK), jnp.int32),
            pltpu.VMEM((CHUNK,), jnp.int32),
            pltpu.VMEM((CHUNK,), jnp.int32),
            pltpu.VMEM((CHUNK // 2,), jnp.int32),
            pltpu.VMEM((CHUNK // 2,), jnp.int32),
            pltpu.VMEM((CHUNK // 2,), jnp.int32),
            pltpu.VMEM((CHUNK // 2,), jnp.int32),
            pltpu.VMEM((CHUNK, D), jnp.float32),
            pltpu.VMEM((CHUNK, D), jnp.float32),
            pltpu.VMEM_SHARED((N_PAD, D), jnp.float32),
            pltpu.SemaphoreType.DMA,
            pltpu.SemaphoreType.DMA,
            pltpu.SemaphoreType.DMA,
            pltpu.SemaphoreType.DMA,
            pltpu.SemaphoreType.DMA,
            pltpu.SemaphoreType.DMA,
        ],
    )
    def agg_kernel(x_hbm, combo_hbm, out_hbm, combo_v, s0, s1,
                   d0a, d0b, d1a, d1b, b0, b1, acc,
                   g0, g1, sa0, sb0, sa1, sb1):
        bufs = (b0, b1)
        srcs = (s0, s1)
        dsts_lo = (d0a, d1a)
        dsts_hi = (d0b, d1b)
        gsems = (g0, g1)
        ssems_lo = (sa0, sa1)
        ssems_hi = (sb0, sb1)
        cid = lax.axis_index("c")
        sid = lax.axis_index("s")
        wid = cid * NS + sid

        row0 = sid * ROWS_PER_TILE

        @pl.when(sid < NS - 1)
        def _():
            pltpu.sync_copy(x_hbm.at[pl.ds(row0, ROWS_PER_TILE)],
                            acc.at[pl.ds(row0, ROWS_PER_TILE)])

        @pl.when(sid == NS - 1)
        def _():
            pltpu.sync_copy(x_hbm.at[pl.ds((NS - 1) * ROWS_PER_TILE, LAST_ROWS)],
                            acc.at[pl.ds((NS - 1) * ROWS_PER_TILE, LAST_ROWS)])

        plsc.subcore_barrier()

        pltpu.sync_copy(combo_hbm.at[pl.ds(wid * K, K)], combo_v)

        def unpack_idx(c, slot):
            half = CHUNK // 32
            for i in range(CHUNK // 16):
                w = combo_v[c, pl.ds(i * 16, 16)]
                srcs[slot][pl.ds(i * 16, 16)] = w & IDX_MASK
                d = lax.shift_right_logical(w, IDX_SHIFT)
                if i < half:
                    dsts_lo[slot][pl.ds(i * 16, 16)] = d
                else:
                    dsts_hi[slot][pl.ds((i - half) * 16, 16)] = d

        H = CHUNK // 2

        def gstart(b):
            pltpu.async_copy(x_hbm.at[srcs[b]], bufs[b], gsems[b])

        def gwait(b):
            pltpu.make_async_copy(x_hbm.at[srcs[b]], bufs[b], gsems[b]).wait()

        def sstart(b):
            pltpu.async_copy(bufs[b].at[pl.ds(0, H)], acc.at[dsts_lo[b]],
                             ssems_lo[b], add=True)
            pltpu.async_copy(bufs[b].at[pl.ds(H, H)], acc.at[dsts_hi[b]],
                             ssems_hi[b], add=True)

        def swait(b):
            pltpu.make_async_copy(bufs[b].at[pl.ds(0, H)], acc.at[dsts_lo[b]],
                                  ssems_lo[b]).wait()
            pltpu.make_async_copy(bufs[b].at[pl.ds(H, H)], acc.at[dsts_hi[b]],
                                  ssems_hi[b]).wait()

        unpack_idx(0, 0)
        gstart(0)
        unpack_idx(1, 1)
        gstart(1)
        gwait(0)
        sstart(0)

        def step(t, b):
            swait(b)
            unpack_idx(t, b)
            gstart(b)
            gwait(1 - b)
            sstart(1 - b)

        def body(m, carry):
            step(2 * m + 2, 0)
            step(2 * m + 3, 1)
            return carry

        lax.fori_loop(0, (K - 2) // 2, body, 0)

        gwait(1)
        sstart(1)
        swait(0)
        swait(1)

        plsc.subcore_barrier()

        pltpu.sync_copy(acc.at[pl.ds(row0, ROWS_PER_TILE)],
                        out_hbm.at[cid, pl.ds(row0, ROWS_PER_TILE)])

    return agg_kernel(x, combo2)


BR = 2000

_PAD_I = np.arange(E_PAD - N_EDGES)
_PAD_COMBO = np.asarray(
    ((N_NODES + _PAD_I % (N_PAD - N_NODES)) << IDX_SHIFT)
    | ((_PAD_I * 13) % N_NODES), dtype=np.int32)


def _mlp_body(p_ref, x_ref, w1_ref, b1_ref, g1_ref, be1_ref, m1_ref, v1_ref,
              w2_ref, b2_ref, g2_ref, be2_ref, m2_ref, v2_ref, w3_ref, b3_ref,
              o_ref):
    s1 = g1_ref[...] * lax.rsqrt(v1_ref[...] + BN_EPS)
    c1 = (b1_ref[...] - m1_ref[...]) * s1 + be1_ref[...]
    s2 = g2_ref[...] * lax.rsqrt(v2_ref[...] + BN_EPS)
    c2 = (b2_ref[...] - m2_ref[...]) * s2 + be2_ref[...]
    h = p_ref[0] + p_ref[1] - x_ref[...]
    h = jnp.maximum(jnp.dot(h, w1_ref[...] * s1,
                            preferred_element_type=jnp.float32) + c1, 0.0)
    h = jnp.maximum(jnp.dot(h, w2_ref[...] * s2,
                            preferred_element_type=jnp.float32) + c2, 0.0)
    o_ref[...] = (jnp.dot(h, w3_ref[...], preferred_element_type=jnp.float32)
                  + b3_ref[...])


def _tc_mlp(parts, x, *weights):
    grid = N_NODES // BR
    wspecs = []
    for w in weights:
        if w.ndim == 2:
            wspecs.append(pl.BlockSpec((D, D), lambda i: (0, 0)))
        else:
            wspecs.append(pl.BlockSpec((1, D), lambda i: (0, 0)))
    return pl.pallas_call(
        _mlp_body,
        grid=(grid,),
        in_specs=[
            pl.BlockSpec((NC, BR, D), lambda i: (0, i, 0)),
            pl.BlockSpec((BR, D), lambda i: (i, 0)),
            *wspecs,
        ],
        out_specs=pl.BlockSpec((BR, D), lambda i: (i, 0)),
        out_shape=jax.ShapeDtypeStruct((N_NODES, D), jnp.float32),
    )(parts, x, *(w.reshape(1, D) if w.ndim == 1 else w for w in weights))


BE = 32768


def _pack_body(e_ref, o_ref):
    e = e_ref[...]
    o_ref[...] = jnp.bitwise_or(
        jnp.left_shift(e[1], IDX_SHIFT), e[0]).reshape(BE // CHUNK, CHUNK)


def _tc_pack(edge_index):
    grid = (N_EDGES + BE - 1) // BE
    return pl.pallas_call(
        _pack_body,
        grid=(grid,),
        in_specs=[pl.BlockSpec((2, BE), lambda i: (0, i))],
        out_specs=pl.BlockSpec((BE // CHUNK, CHUNK), lambda i: (i, 0)),
        out_shape=jax.ShapeDtypeStruct((N_EDGES // CHUNK, CHUNK), jnp.int32),
    )(edge_index)


def kernel(x, edge_index, W1, b1, g1, be1, m1, v1, W2, b2, g2, be2, m2, v2, W3, b3):
    combo = _tc_pack(edge_index.astype(jnp.int32))
    combo2 = jnp.concatenate(
        [combo, jnp.asarray(_PAD_COMBO).reshape(-1, CHUNK)])

    parts = _sc_aggregate(x, combo2)

    return _tc_mlp(parts, x, W1, b1, g1, be1, m1, v1,
                   W2, b2, g2, be2, m2, v2, W3, b3)


# speedup vs baseline: 13.6598x; 1.0101x over previous
import functools

import jax
import jax.numpy as jnp
import numpy as np
from jax import lax
from jax.experimental import pallas as pl
from jax.experimental.pallas import tpu as pltpu
from jax.experimental.pallas import tpu_sc as plsc

N_NODES = 10000
D = 128
N_EDGES = 320000
BN_EPS = 1e-5

NC = 2
NS = 16
NW = NC * NS

CHUNK = 128
K = 80
E_PAD = NW * K * CHUNK
ROWS_PER_TILE = 632
LAST_ROWS = N_NODES - 15 * ROWS_PER_TILE
N_PAD = NS * ROWS_PER_TILE
IDX_SHIFT = 14
IDX_MASK = (1 << IDX_SHIFT) - 1


def _sc_aggregate(x, combo2):
    mesh = plsc.VectorSubcoreMesh(core_axis_name="c", subcore_axis_name="s")

    @functools.partial(
        pl.kernel,
        mesh=mesh,
        out_type=jax.ShapeDtypeStruct((NC, N_PAD, D), jnp.float32),
        scratch_types=[
            pltpu.VMEM((K, CHUNK), jnp.int32),
            pltpu.VMEM((CHUNK,), jnp.int32),
            pltpu.VMEM((CHUNK,), jnp.int32),
            pltpu.VMEM((CHUNK // 2,), jnp.int32),
            pltpu.VMEM((CHUNK // 2,), jnp.int32),
            pltpu.VMEM((CHUNK // 2,), jnp.int32),
            pltpu.VMEM((CHUNK // 2,), jnp.int32),
            pltpu.VMEM((CHUNK, D), jnp.float32),
            pltpu.VMEM((CHUNK, D), jnp.float32),
            pltpu.VMEM_SHARED((N_PAD, D), jnp.float32),
            pltpu.SemaphoreType.DMA,
            pltpu.SemaphoreType.DMA,
            pltpu.SemaphoreType.DMA,
            pltpu.SemaphoreType.DMA,
            pltpu.SemaphoreType.DMA,
            pltpu.SemaphoreType.DMA,
        ],
    )
    def agg_kernel(x_hbm, combo_hbm, out_hbm, combo_v, s0, s1,
                   d0a, d0b, d1a, d1b, b0, b1, acc,
                   g0, g1, sa0, sb0, sa1, sb1):
        bufs = (b0, b1)
        srcs = (s0, s1)
        dsts_lo = (d0a, d1a)
        dsts_hi = (d0b, d1b)
        gsems = (g0, g1)
        ssems_lo = (sa0, sa1)
        ssems_hi = (sb0, sb1)
        cid = lax.axis_index("c")
        sid = lax.axis_index("s")
        wid = cid * NS + sid

        row0 = sid * ROWS_PER_TILE

        @pl.when(sid < NS - 1)
        def _():
            pltpu.sync_copy(x_hbm.at[pl.ds(row0, ROWS_PER_TILE)],
                            acc.at[pl.ds(row0, ROWS_PER_TILE)])

        @pl.when(sid == NS - 1)
        def _():
            pltpu.sync_copy(x_hbm.at[pl.ds((NS - 1) * ROWS_PER_TILE, LAST_ROWS)],
                            acc.at[pl.ds((NS - 1) * ROWS_PER_TILE, LAST_ROWS)])

        plsc.subcore_barrier()

        pltpu.sync_copy(combo_hbm.at[pl.ds(wid * K, K)], combo_v)

        def unpack_idx(c, slot):
            half = CHUNK // 32
            for i in range(CHUNK // 16):
                w = combo_v[c, pl.ds(i * 16, 16)]
                srcs[slot][pl.ds(i * 16, 16)] = w & IDX_MASK
                d = lax.shift_right_logical(w, IDX_SHIFT)
                if i < half:
                    dsts_lo[slot][pl.ds(i * 16, 16)] = d
                else:
                    dsts_hi[slot][pl.ds((i - half) * 16, 16)] = d

        H = CHUNK // 2

        def gstart(b):
            pltpu.async_copy(x_hbm.at[srcs[b]], bufs[b], gsems[b])

        def gwait(b):
            pltpu.make_async_copy(x_hbm.at[srcs[b]], bufs[b], gsems[b]).wait()

        def sstart(b):
            pltpu.async_copy(bufs[b].at[pl.ds(0, H)], acc.at[dsts_lo[b]],
                             ssems_lo[b], add=True)
            pltpu.async_copy(bufs[b].at[pl.ds(H, H)], acc.at[dsts_hi[b]],
                             ssems_hi[b], add=True)

        def swait(b):
            pltpu.make_async_copy(bufs[b].at[pl.ds(0, H)], acc.at[dsts_lo[b]],
                                  ssems_lo[b]).wait()
            pltpu.make_async_copy(bufs[b].at[pl.ds(H, H)], acc.at[dsts_hi[b]],
                                  ssems_hi[b]).wait()

        unpack_idx(0, 0)
        gstart(0)
        unpack_idx(1, 1)
        gstart(1)
        gwait(0)
        sstart(0)

        def step(t, b):
            swait(b)
            unpack_idx(t, b)
            gstart(b)
            gwait(1 - b)
            sstart(1 - b)

        def body(m, carry):
            step(2 * m + 2, 0)
            step(2 * m + 3, 1)
            return carry

        lax.fori_loop(0, (K - 2) // 2, body, 0)

        gwait(1)
        sstart(1)
        swait(0)
        swait(1)

        plsc.subcore_barrier()

        pltpu.sync_copy(acc.at[pl.ds(row0, ROWS_PER_TILE)],
                        out_hbm.at[cid, pl.ds(row0, ROWS_PER_TILE)])

    return agg_kernel(x, combo2)


BR = 2000


def _mlp_body(p_ref, x_ref, w1_ref, b1_ref, g1_ref, be1_ref, m1_ref, v1_ref,
              w2_ref, b2_ref, g2_ref, be2_ref, m2_ref, v2_ref, w3_ref, b3_ref,
              o_ref):
    s1 = g1_ref[...] * lax.rsqrt(v1_ref[...] + BN_EPS)
    c1 = (b1_ref[...] - m1_ref[...]) * s1 + be1_ref[...]
    s2 = g2_ref[...] * lax.rsqrt(v2_ref[...] + BN_EPS)
    c2 = (b2_ref[...] - m2_ref[...]) * s2 + be2_ref[...]
    h = p_ref[0] + p_ref[1] - x_ref[...]
    h = jnp.maximum(jnp.dot(h, w1_ref[...] * s1,
                            preferred_element_type=jnp.float32) + c1, 0.0)
    h = jnp.maximum(jnp.dot(h, w2_ref[...] * s2,
                            preferred_element_type=jnp.float32) + c2, 0.0)
    o_ref[...] = (jnp.dot(h, w3_ref[...], preferred_element_type=jnp.float32)
                  + b3_ref[...])


def _tc_mlp(parts, x, *weights):
    grid = N_NODES // BR
    wspecs = []
    for w in weights:
        if w.ndim == 2:
            wspecs.append(pl.BlockSpec((D, D), lambda i: (0, 0)))
        else:
            wspecs.append(pl.BlockSpec((1, D), lambda i: (0, 0)))
    return pl.pallas_call(
        _mlp_body,
        grid=(grid,),
        in_specs=[
            pl.BlockSpec((NC, BR, D), lambda i: (0, i, 0)),
            pl.BlockSpec((BR, D), lambda i: (i, 0)),
            *wspecs,
        ],
        out_specs=pl.BlockSpec((BR, D), lambda i: (i, 0)),
        out_shape=jax.ShapeDtypeStruct((N_NODES, D), jnp.float32),
    )(parts, x, *(w.reshape(1, D) if w.ndim == 1 else w for w in weights))


BE = 32768


def _pack_body(e_ref, o_ref):
    i = pl.program_id(0)
    e = e_ref[...]
    packed = jnp.bitwise_or(jnp.left_shift(e[1], IDX_SHIFT),
                            e[0]).reshape(BE // CHUNK, CHUNK)
    flat = (i * BE
            + jax.lax.broadcasted_iota(jnp.int32, (BE // CHUNK, CHUNK), 0)
            * CHUNK
            + jax.lax.broadcasted_iota(jnp.int32, (BE // CHUNK, CHUNK), 1))
    p = jnp.maximum(flat - N_EDGES, 0)
    padv = jnp.bitwise_or(
        jnp.left_shift(N_NODES + p % (N_PAD - N_NODES), IDX_SHIFT),
        (p * 13) % N_NODES)
    o_ref[...] = jnp.where(flat < N_EDGES, packed, padv)


def _tc_pack(edge_index):
    grid = E_PAD // BE
    return pl.pallas_call(
        _pack_body,
        grid=(grid,),
        in_specs=[pl.BlockSpec((2, BE), lambda i: (0, i))],
        out_specs=pl.BlockSpec((BE // CHUNK, CHUNK), lambda i: (i, 0)),
        out_shape=jax.ShapeDtypeStruct((E_PAD // CHUNK, CHUNK), jnp.int32),
    )(edge_index)


def kernel(x, edge_index, W1, b1, g1, be1, m1, v1, W2, b2, g2, be2, m2, v2, W3, b3):
    combo2 = _tc_pack(edge_index.astype(jnp.int32))

    parts = _sc_aggregate(x, combo2)

    return _tc_mlp(parts, x, W1, b1, g1, be1, m1, v1,
                   W2, b2, g2, be2, m2, v2, W3, b3)


# speedup vs baseline: 13.7023x; 1.0031x over previous
import functools

import jax
import jax.numpy as jnp
import numpy as np
from jax import lax
from jax.experimental import pallas as pl
from jax.experimental.pallas import tpu as pltpu
from jax.experimental.pallas import tpu_sc as plsc

N_NODES = 10000
D = 128
N_EDGES = 320000
BN_EPS = 1e-5

NC = 2
NS = 16
NW = NC * NS

CHUNK = 128
K = 80
E_PAD = NW * K * CHUNK
ROWS_PER_TILE = 632
LAST_ROWS = N_NODES - 15 * ROWS_PER_TILE
N_PAD = NS * ROWS_PER_TILE
IDX_SHIFT = 14
IDX_MASK = (1 << IDX_SHIFT) - 1


def _sc_aggregate(x, combo2):
    mesh = plsc.VectorSubcoreMesh(core_axis_name="c", subcore_axis_name="s")

    @functools.partial(
        pl.kernel,
        mesh=mesh,
        out_type=jax.ShapeDtypeStruct((NC, N_PAD, D), jnp.float32),
        scratch_types=[
            pltpu.VMEM((K, CHUNK), jnp.int32),
            pltpu.VMEM((CHUNK,), jnp.int32),
            pltpu.VMEM((CHUNK,), jnp.int32),
            pltpu.VMEM((CHUNK // 2,), jnp.int32),
            pltpu.VMEM((CHUNK // 2,), jnp.int32),
            pltpu.VMEM((CHUNK // 2,), jnp.int32),
            pltpu.VMEM((CHUNK // 2,), jnp.int32),
            pltpu.VMEM((CHUNK, D), jnp.float32),
            pltpu.VMEM((CHUNK, D), jnp.float32),
            pltpu.VMEM_SHARED((N_PAD, D), jnp.float32),
            pltpu.SemaphoreType.DMA,
            pltpu.SemaphoreType.DMA,
            pltpu.SemaphoreType.DMA,
            pltpu.SemaphoreType.DMA,
            pltpu.SemaphoreType.DMA,
            pltpu.SemaphoreType.DMA,
        ],
    )
    def agg_kernel(x_hbm, combo_hbm, zero_hbm, out_hbm, combo_v, s0, s1,
                   d0a, d0b, d1a, d1b, b0, b1, acc,
                   g0, g1, sa0, sb0, sa1, sb1):
        bufs = (b0, b1)
        srcs = (s0, s1)
        dsts_lo = (d0a, d1a)
        dsts_hi = (d0b, d1b)
        gsems = (g0, g1)
        ssems_lo = (sa0, sa1)
        ssems_hi = (sb0, sb1)
        cid = lax.axis_index("c")
        sid = lax.axis_index("s")
        wid = cid * NS + sid

        pltpu.sync_copy(combo_hbm.at[pl.ds(wid * K, K)], combo_v)

        row0 = sid * ROWS_PER_TILE

        @pl.when(jnp.logical_and(cid == 1, sid < NS - 1))
        def _():
            pltpu.sync_copy(x_hbm.at[pl.ds(row0, ROWS_PER_TILE)],
                            acc.at[pl.ds(row0, ROWS_PER_TILE)])

        @pl.when(jnp.logical_and(cid == 1, sid == NS - 1))
        def _():
            pltpu.sync_copy(x_hbm.at[pl.ds((NS - 1) * ROWS_PER_TILE, LAST_ROWS)],
                            acc.at[pl.ds((NS - 1) * ROWS_PER_TILE, LAST_ROWS)])

        @pl.when(jnp.logical_and(cid == 0, sid < NS - 1))
        def _():
            pltpu.sync_copy(zero_hbm, acc.at[pl.ds(row0, ROWS_PER_TILE)])

        @pl.when(jnp.logical_and(cid == 0, sid == NS - 1))
        def _():
            pltpu.sync_copy(zero_hbm.at[pl.ds(0, LAST_ROWS)],
                            acc.at[pl.ds((NS - 1) * ROWS_PER_TILE, LAST_ROWS)])

        plsc.subcore_barrier()

        def unpack_idx(c, slot):
            half = CHUNK // 32
            for i in range(CHUNK // 16):
                w = combo_v[c, pl.ds(i * 16, 16)]
                srcs[slot][pl.ds(i * 16, 16)] = w & IDX_MASK
                d = lax.shift_right_logical(w, IDX_SHIFT)
                if i < half:
                    dsts_lo[slot][pl.ds(i * 16, 16)] = d
                else:
                    dsts_hi[slot][pl.ds((i - half) * 16, 16)] = d

        H = CHUNK // 2

        def gstart(b):
            pltpu.async_copy(x_hbm.at[srcs[b]], bufs[b], gsems[b])

        def gwait(b):
            pltpu.make_async_copy(x_hbm.at[srcs[b]], bufs[b], gsems[b]).wait()

        def sstart(b):
            pltpu.async_copy(bufs[b].at[pl.ds(0, H)], acc.at[dsts_lo[b]],
                             ssems_lo[b], add=True)
            pltpu.async_copy(bufs[b].at[pl.ds(H, H)], acc.at[dsts_hi[b]],
                             ssems_hi[b], add=True)

        def swait(b):
            pltpu.make_async_copy(bufs[b].at[pl.ds(0, H)], acc.at[dsts_lo[b]],
                                  ssems_lo[b]).wait()
            pltpu.make_async_copy(bufs[b].at[pl.ds(H, H)], acc.at[dsts_hi[b]],
                                  ssems_hi[b]).wait()

        unpack_idx(0, 0)
        gstart(0)
        unpack_idx(1, 1)
        gstart(1)
        gwait(0)
        sstart(0)

        def step(t, b):
            swait(b)
            unpack_idx(t, b)
            gstart(b)
            gwait(1 - b)
            sstart(1 - b)

        def body(m, carry):
            step(2 * m + 2, 0)
            step(2 * m + 3, 1)
            return carry

        lax.fori_loop(0, (K - 2) // 2, body, 0)

        gwait(1)
        sstart(1)
        swait(0)
        swait(1)

        plsc.subcore_barrier()

        pltpu.sync_copy(acc.at[pl.ds(row0, ROWS_PER_TILE)],
                        out_hbm.at[cid, pl.ds(row0, ROWS_PER_TILE)])

    return agg_kernel(x, combo2, jnp.zeros((ROWS_PER_TILE, D), jnp.float32))


BR = 2000


def _mlp_body(p_ref, w1_ref, b1_ref, g1_ref, be1_ref, m1_ref, v1_ref,
              w2_ref, b2_ref, g2_ref, be2_ref, m2_ref, v2_ref, w3_ref, b3_ref,
              o_ref):
    s1 = g1_ref[...] * lax.rsqrt(v1_ref[...] + BN_EPS)
    c1 = (b1_ref[...] - m1_ref[...]) * s1 + be1_ref[...]
    s2 = g2_ref[...] * lax.rsqrt(v2_ref[...] + BN_EPS)
    c2 = (b2_ref[...] - m2_ref[...]) * s2 + be2_ref[...]
    h = p_ref[0] + p_ref[1]
    h = jnp.maximum(jnp.dot(h, w1_ref[...] * s1,
                            preferred_element_type=jnp.float32) + c1, 0.0)
    h = jnp.maximum(jnp.dot(h, w2_ref[...] * s2,
                            preferred_element_type=jnp.float32) + c2, 0.0)
    o_ref[...] = (jnp.dot(h, w3_ref[...], preferred_element_type=jnp.float32)
                  + b3_ref[...])


def _tc_mlp(parts, *weights):
    grid = N_NODES // BR
    wspecs = []
    for w in weights:
        if w.ndim == 2:
            wspecs.append(pl.BlockSpec((D, D), lambda i: (0, 0)))
        else:
            wspecs.append(pl.BlockSpec((1, D), lambda i: (0, 0)))
    return pl.pallas_call(
        _mlp_body,
        grid=(grid,),
        in_specs=[
            pl.BlockSpec((NC, BR, D), lambda i: (0, i, 0)),
            *wspecs,
        ],
        out_specs=pl.BlockSpec((BR, D), lambda i: (i, 0)),
        out_shape=jax.ShapeDtypeStruct((N_NODES, D), jnp.float32),
    )(parts, *(w.reshape(1, D) if w.ndim == 1 else w for w in weights))


BE = 32768


def _pack_body(e_ref, o_ref):
    i = pl.program_id(0)
    e = e_ref[...]
    packed = jnp.bitwise_or(jnp.left_shift(e[1], IDX_SHIFT),
                            e[0]).reshape(BE // CHUNK, CHUNK)
    flat = (i * BE
            + jax.lax.broadcasted_iota(jnp.int32, (BE // CHUNK, CHUNK), 0)
            * CHUNK
            + jax.lax.broadcasted_iota(jnp.int32, (BE // CHUNK, CHUNK), 1))
    p = jnp.maximum(flat - N_EDGES, 0)
    padv = jnp.bitwise_or(
        jnp.left_shift(N_NODES + p % (N_PAD - N_NODES), IDX_SHIFT),
        (p * 13) % N_NODES)
    o_ref[...] = jnp.where(flat < N_EDGES, packed, padv)


def _tc_pack(edge_index):
    grid = E_PAD // BE
    return pl.pallas_call(
        _pack_body,
        grid=(grid,),
        in_specs=[pl.BlockSpec((2, BE), lambda i: (0, i))],
        out_specs=pl.BlockSpec((BE // CHUNK, CHUNK), lambda i: (i, 0)),
        out_shape=jax.ShapeDtypeStruct((E_PAD // CHUNK, CHUNK), jnp.int32),
    )(edge_index)


def kernel(x, edge_index, W1, b1, g1, be1, m1, v1, W2, b2, g2, be2, m2, v2, W3, b3):
    combo2 = _tc_pack(edge_index.astype(jnp.int32))

    parts = _sc_aggregate(x, combo2)

    return _tc_mlp(parts, W1, b1, g1, be1, m1, v1,
                   W2, b2, g2, be2, m2, v2, W3, b3)


# speedup vs baseline: 13.7087x; 1.0005x over previous
import functools

import jax
import jax.numpy as jnp
from jax import lax
from jax.experimental import pallas as pl
from jax.experimental.pallas import tpu as pltpu
from jax.experimental.pallas import tpu_sc as plsc

N_NODES = 10000
D = 128
N_EDGES = 320000
BN_EPS = 1e-5

NC = 2
NS = 16
NW = NC * NS

CHUNK = 128
K = 80
E_PAD = NW * K * CHUNK
ROWS_PER_TILE = 632
LAST_ROWS = N_NODES - 15 * ROWS_PER_TILE
N_PAD = NS * ROWS_PER_TILE
IDX_SHIFT = 14
IDX_MASK = (1 << IDX_SHIFT) - 1


def _sc_aggregate(x, combo2):
    mesh = plsc.VectorSubcoreMesh(core_axis_name="c", subcore_axis_name="s")

    @functools.partial(
        pl.kernel,
        mesh=mesh,
        out_type=jax.ShapeDtypeStruct((NC, N_PAD, D), jnp.float32),
        scratch_types=[
            pltpu.VMEM((K, CHUNK), jnp.int32),
            pltpu.VMEM((CHUNK,), jnp.int32),
            pltpu.VMEM((CHUNK,), jnp.int32),
            pltpu.VMEM((CHUNK // 2,), jnp.int32),
            pltpu.VMEM((CHUNK // 2,), jnp.int32),
            pltpu.VMEM((CHUNK // 2,), jnp.int32),
            pltpu.VMEM((CHUNK // 2,), jnp.int32),
            pltpu.VMEM((CHUNK, D), jnp.float32),
            pltpu.VMEM((CHUNK, D), jnp.float32),
            pltpu.VMEM_SHARED((N_PAD, D), jnp.float32),
            pltpu.SemaphoreType.DMA,
            pltpu.SemaphoreType.DMA,
            pltpu.SemaphoreType.DMA,
            pltpu.SemaphoreType.DMA,
            pltpu.SemaphoreType.DMA,
            pltpu.SemaphoreType.DMA,
        ],
    )
    def agg_kernel(x_hbm, combo_hbm, zero_hbm, out_hbm, combo_v, s0, s1,
                   d0a, d0b, d1a, d1b, b0, b1, acc,
                   g0, g1, sa0, sb0, sa1, sb1):
        bufs = (b0, b1)
        srcs = (s0, s1)
        dsts_lo = (d0a, d1a)
        dsts_hi = (d0b, d1b)
        gsems = (g0, g1)
        ssems_lo = (sa0, sa1)
        ssems_hi = (sb0, sb1)
        cid = lax.axis_index("c")
        sid = lax.axis_index("s")
        wid = cid * NS + sid

        pltpu.sync_copy(combo_hbm.at[pl.ds(wid * K, K)], combo_v)

        row0 = sid * ROWS_PER_TILE

        @pl.when(jnp.logical_and(cid == 1, sid < NS - 1))
        def _():
            pltpu.sync_copy(x_hbm.at[pl.ds(row0, ROWS_PER_TILE)],
                            acc.at[pl.ds(row0, ROWS_PER_TILE)])

        @pl.when(jnp.logical_and(cid == 1, sid == NS - 1))
        def _():
            pltpu.sync_copy(x_hbm.at[pl.ds((NS - 1) * ROWS_PER_TILE, LAST_ROWS)],
                            acc.at[pl.ds((NS - 1) * ROWS_PER_TILE, LAST_ROWS)])

        @pl.when(jnp.logical_and(cid == 0, sid < NS - 1))
        def _():
            pltpu.sync_copy(zero_hbm, acc.at[pl.ds(row0, ROWS_PER_TILE)])

        @pl.when(jnp.logical_and(cid == 0, sid == NS - 1))
        def _():
            pltpu.sync_copy(zero_hbm.at[pl.ds(0, LAST_ROWS)],
                            acc.at[pl.ds((NS - 1) * ROWS_PER_TILE, LAST_ROWS)])

        plsc.subcore_barrier()

        def unpack_idx(c, slot):
            half = CHUNK // 32
            for i in range(CHUNK // 16):
                w = combo_v[c, pl.ds(i * 16, 16)]
                srcs[slot][pl.ds(i * 16, 16)] = w & IDX_MASK
                d = lax.shift_right_logical(w, IDX_SHIFT)
                if i < half:
                    dsts_lo[slot][pl.ds(i * 16, 16)] = d
                else:
                    dsts_hi[slot][pl.ds((i - half) * 16, 16)] = d

        H = CHUNK // 2

        def gstart(b):
            pltpu.async_copy(x_hbm.at[srcs[b]], bufs[b], gsems[b])

        def gwait(b):
            pltpu.make_async_copy(x_hbm.at[srcs[b]], bufs[b], gsems[b]).wait()

        def sstart(b):
            pltpu.async_copy(bufs[b].at[pl.ds(0, H)], acc.at[dsts_lo[b]],
                             ssems_lo[b], add=True)
            pltpu.async_copy(bufs[b].at[pl.ds(H, H)], acc.at[dsts_hi[b]],
                             ssems_hi[b], add=True)

        def swait(b):
            pltpu.make_async_copy(bufs[b].at[pl.ds(0, H)], acc.at[dsts_lo[b]],
                                  ssems_lo[b]).wait()
            pltpu.make_async_copy(bufs[b].at[pl.ds(H, H)], acc.at[dsts_hi[b]],
                                  ssems_hi[b]).wait()

        unpack_idx(0, 0)
        gstart(0)
        unpack_idx(1, 1)
        gstart(1)
        gwait(0)
        sstart(0)

        def step(t, b):
            swait(b)
            unpack_idx(t, b)
            gstart(b)
            gwait(1 - b)
            sstart(1 - b)

        def body(m, carry):
            step(2 * m + 2, 0)
            step(2 * m + 3, 1)
            return carry

        lax.fori_loop(0, (K - 2) // 2, body, 0)

        gwait(1)
        sstart(1)
        swait(0)
        swait(1)

        plsc.subcore_barrier()

        pltpu.sync_copy(acc.at[pl.ds(row0, ROWS_PER_TILE)],
                        out_hbm.at[cid, pl.ds(row0, ROWS_PER_TILE)])

    return agg_kernel(x, combo2, jnp.zeros((ROWS_PER_TILE, D), jnp.float32))


BR = 2000


def _mlp_body(p_ref, w1_ref, b1_ref, g1_ref, be1_ref, m1_ref, v1_ref,
              w2_ref, b2_ref, g2_ref, be2_ref, m2_ref, v2_ref, w3_ref, b3_ref,
              o_ref):
    s1 = g1_ref[...] * lax.rsqrt(v1_ref[...] + BN_EPS)
    c1 = (b1_ref[...] - m1_ref[...]) * s1 + be1_ref[...]
    s2 = g2_ref[...] * lax.rsqrt(v2_ref[...] + BN_EPS)
    c2 = (b2_ref[...] - m2_ref[...]) * s2 + be2_ref[...]
    h = p_ref[0] + p_ref[1]
    h = jnp.maximum(jnp.dot(h, w1_ref[...] * s1,
                            preferred_element_type=jnp.float32) + c1, 0.0)
    h = jnp.maximum(jnp.dot(h, w2_ref[...] * s2,
                            preferred_element_type=jnp.float32) + c2, 0.0)
    o_ref[...] = (jnp.dot(h, w3_ref[...], preferred_element_type=jnp.float32)
                  + b3_ref[...])


def _tc_mlp(parts, *weights):
    grid = N_NODES // BR
    wspecs = []
    for w in weights:
        if w.ndim == 2:
            wspecs.append(pl.BlockSpec((D, D), lambda i: (0, 0)))
        else:
            wspecs.append(pl.BlockSpec((1, D), lambda i: (0, 0)))
    return pl.pallas_call(
        _mlp_body,
        grid=(grid,),
        in_specs=[
            pl.BlockSpec((NC, BR, D), lambda i: (0, i, 0)),
            *wspecs,
        ],
        out_specs=pl.BlockSpec((BR, D), lambda i: (i, 0)),
        out_shape=jax.ShapeDtypeStruct((N_NODES, D), jnp.float32),
    )(parts, *(w.reshape(1, D) if w.ndim == 1 else w for w in weights))


BE = 32768


def _pack_body(e_ref, o_ref):
    i = pl.program_id(0)
    e = e_ref[...]
    packed = jnp.bitwise_or(jnp.left_shift(e[1], IDX_SHIFT),
                            e[0]).reshape(BE // CHUNK, CHUNK)
    flat = (i * BE
            + jax.lax.broadcasted_iota(jnp.int32, (BE // CHUNK, CHUNK), 0)
            * CHUNK
            + jax.lax.broadcasted_iota(jnp.int32, (BE // CHUNK, CHUNK), 1))
    p = jnp.maximum(flat - N_EDGES, 0)
    padv = jnp.bitwise_or(
        jnp.left_shift(N_NODES + p % (N_PAD - N_NODES), IDX_SHIFT),
        (p * 13) % N_NODES)
    o_ref[...] = jnp.where(flat < N_EDGES, packed, padv)


def _tc_pack(edge_index):
    grid = E_PAD // BE
    return pl.pallas_call(
        _pack_body,
        grid=(grid,),
        in_specs=[pl.BlockSpec((2, BE), lambda i: (0, i))],
        out_specs=pl.BlockSpec((BE // CHUNK, CHUNK), lambda i: (i, 0)),
        out_shape=jax.ShapeDtypeStruct((E_PAD // CHUNK, CHUNK), jnp.int32),
    )(edge_index)


def kernel(x, edge_index, W1, b1, g1, be1, m1, v1, W2, b2, g2, be2, m2, v2, W3, b3):
    combo2 = _tc_pack(edge_index.astype(jnp.int32))

    parts = _sc_aggregate(x, combo2)

    return _tc_mlp(parts, W1, b1, g1, be1, m1, v1,
                   W2, b2, g2, be2, m2, v2, W3, b3)
